# baseline (speedup 1.0000x reference)
import functools

import jax
import jax.numpy as jnp
from jax import lax
from jax.experimental import pallas as pl
from jax.experimental.pallas import tpu as pltpu
from jax.experimental.pallas import tpu_sc as plsc

B, N, M, H, E = 4, 10000, 128, 32, 160000
NC, NS = 2, 16
NW = NC * NS
EPW = E // NW
CH = 128
NCHK = 40
EPWP = NCHK * CH
NBUF = 4
def _chunking(dout):
    ch, nbuf = (40, 1) if dout > 64 else (128, 4)
    return ch, EPWP // ch, nbuf
NPAD = 10112
RPT = NPAD // NS
DOUTS = [H, 2 * H, 4 * H, 4 * H, 4 * H]

_MESH = plsc.VectorSubcoreMesh(
    core_axis_name="c", subcore_axis_name="s", num_cores=NC, num_subcores=NS)
_SC_PARAMS = pltpu.CompilerParams(use_tc_tiling_on_sc=False)


def _sc_deg_body(ei_hbm, ones_hbm, zeros_hbm, out_hbm, slab, ones_v, acc,
                 sem):
    c = lax.axis_index("c")
    s = lax.axis_index("s")
    w = c * NS + s
    r0 = s * RPT
    pltpu.sync_copy(ones_hbm, ones_v)
    for b in range(B):
        pltpu.sync_copy(zeros_hbm.at[pl.ds(r0, RPT)], acc.at[pl.ds(r0, RPT)])
        plsc.subcore_barrier()
        pltpu.sync_copy(ei_hbm.at[b, w], slab)

        def fire(j, carry):
            pltpu.async_copy(ones_v, acc.at[slab.at[j, 1]], sem, add=True)
            return carry

        lax.fori_loop(0, NCHK, fire, 0)

        def drain(j, carry):
            pltpu.make_async_copy(ones_v, acc.at[slab.at[j, 1]], sem).wait()
            return carry

        lax.fori_loop(0, NCHK, drain, 0)
        plsc.subcore_barrier()
        pltpu.sync_copy(acc.at[pl.ds(r0, RPT)],
                        out_hbm.at[b, c, pl.ds(r0, RPT)])


_sc_deg = pl.kernel(
    _sc_deg_body,
    out_type=jax.ShapeDtypeStruct((B, NC, NPAD, 16), jnp.float32),
    mesh=_MESH,
    scratch_types=[
        pltpu.VMEM((NCHK, 2, CH), jnp.int32),
        pltpu.VMEM((CH, 16), jnp.float32),
        pltpu.VMEM_SHARED((NPAD, 16), jnp.float32),
        pltpu.SemaphoreType.DMA,
    ],
    compiler_params=_SC_PARAMS,
)


def _sc_agg_body(dout, g_hbm, ei_hbm, zeros_hbm, out_hbm, slab, rows, acc,
                 gsem, ssem):
    _, nchk, nbuf = _chunking(dout)
    lk = nbuf // 2
    c = lax.axis_index("c")
    s = lax.axis_index("s")
    w = c * NS + s
    r0 = s * RPT

    if nbuf == 1:
        def sync_step(j, carry):
            pltpu.sync_copy(g_hbm.at[slab.at[j, 0]], rows[0])
            pltpu.sync_copy(rows[0], acc.at[slab.at[j, 1]], add=True)
            return carry

    def step(j, k):
        k2 = (k + lk) % nbuf
        pltpu.make_async_copy(g_hbm.at[slab.at[j, 0]], rows[k], gsem[k]).wait()
        pltpu.async_copy(rows[k], acc.at[slab.at[j, 1]], ssem[k], add=True)

        @pl.when(j >= lk)
        def _():
            pltpu.make_async_copy(rows[k2], acc.at[slab.at[j - lk, 1]],
                                  ssem[k2]).wait()

        @pl.when(j < nchk - lk)
        def _():
            pltpu.async_copy(g_hbm.at[slab.at[j + lk, 0]], rows[k2], gsem[k2])

    for b in range(B):
        pltpu.sync_copy(zeros_hbm.at[pl.ds(r0, RPT)], acc.at[pl.ds(r0, RPT)])
        plsc.subcore_barrier()
        pltpu.sync_copy(ei_hbm.at[b, w], slab)
        if nbuf == 1:
            lax.fori_loop(0, nchk, sync_step, 0)
        else:
            for t in range(lk):
                pltpu.async_copy(g_hbm.at[slab.at[t, 0]], rows[t], gsem[t])

            def grp(m, carry):
                for k in range(nbuf):
                    step(nbuf * m + k, k)
                return carry

            lax.fori_loop(0, nchk // nbuf, grp, 0)
            for t in range(lk):
                j = nchk - lk + t
                pltpu.make_async_copy(rows[j % nbuf], acc.at[slab.at[j, 1]],
                                      ssem[j % nbuf]).wait()
        plsc.subcore_barrier()
        pltpu.sync_copy(acc.at[pl.ds(r0, RPT)],
                        out_hbm.at[b, c, pl.ds(r0, RPT)])


@functools.cache
def _sc_agg(dout):
    ch, nchk, nbuf = _chunking(dout)
    return pl.kernel(
        functools.partial(_sc_agg_body, dout),
        out_type=jax.ShapeDtypeStruct((B, NC, NPAD, dout), jnp.float32),
        mesh=_MESH,
        scratch_types=[
            pltpu.VMEM((nchk, 2, ch), jnp.int32),
            [pltpu.VMEM((ch, dout), jnp.float32)] * nbuf,
            pltpu.VMEM_SHARED((NPAD, dout), jnp.float32),
            [pltpu.SemaphoreType.DMA] * nbuf,
            [pltpu.SemaphoreType.DMA] * nbuf,
        ],
        compiler_params=_SC_PARAMS,
    )


RC = 5
CHR = N // RC


def _tc_dinv_body(deg_ref, out_ref):
    deg = deg_ref[0, 0, :, 0:1] + deg_ref[0, 1, :, 0:1] + 1.0
    out_ref[0] = lax.rsqrt(deg)


def _tc_dinv(deg):
    return pl.pallas_call(
        _tc_dinv_body,
        grid=(B,),
        in_specs=[pl.BlockSpec((1, NC, NPAD, 16), lambda b: (b, 0, 0, 0))],
        out_specs=pl.BlockSpec((1, NPAD, 1), lambda b: (b, 0, 0)),
        out_shape=jax.ShapeDtypeStruct((B, NPAD, 1), jnp.float32),
    )(deg)


def _tc_first_body(x_ref, dinv_ref, w_ref, g_ref):
    hw = jnp.dot(x_ref[0], w_ref[...], preferred_element_type=jnp.float32)
    g_ref[0] = dinv_ref[0] * hw


def _tc_first(xs, dinv, w1):
    return pl.pallas_call(
        _tc_first_body,
        grid=(B, RC),
        in_specs=[pl.BlockSpec((1, CHR, M), lambda b, r: (b, r, 0)),
                  pl.BlockSpec((1, CHR, 1), lambda b, r: (b, r, 0)),
                  pl.BlockSpec((M, H), lambda b, r: (0, 0))],
        out_specs=pl.BlockSpec((1, CHR, H), lambda b, r: (b, r, 0)),
        out_shape=jax.ShapeDtypeStruct((B, N, H), jnp.float32),
    )(xs, dinv, w1)


def _tc_act_body(s_ref, g_ref, dinv_ref, b_ref, act_ref, st_ref):
    tot = s_ref[0, 0] + s_ref[0, 1] + g_ref[0]
    pre = dinv_ref[0] * tot + b_ref[...]
    act = jnp.where(pre >= 0, pre, 0.01 * pre)
    act_ref[0] = act
    st_ref[0, 0, 0] = jnp.sum(act, axis=0)
    st_ref[0, 0, 1] = jnp.sum(act * act, axis=0)


def _tc_act(s, g, dinv, bi, dout):
    return pl.pallas_call(
        _tc_act_body,
        grid=(B, RC),
        in_specs=[pl.BlockSpec((1, NC, CHR, dout), lambda b, r: (b, 0, r, 0)),
                  pl.BlockSpec((1, CHR, dout), lambda b, r: (b, r, 0)),
                  pl.BlockSpec((1, CHR, 1), lambda b, r: (b, r, 0)),
                  pl.BlockSpec((1, dout), lambda b, r: (0, 0))],
        out_specs=[pl.BlockSpec((1, CHR, dout), lambda b, r: (b, r, 0)),
                   pl.BlockSpec((1, 1, 2, dout), lambda b, r: (b, r, 0, 0))],
        out_shape=[jax.ShapeDtypeStruct((B, N, dout), jnp.float32),
                   jax.ShapeDtypeStruct((B, RC, 2, dout), jnp.float32)],
    )(s, g, dinv, bi)


def _bn(act_ref, st_ref, bng_ref, bnb_ref):
    st = st_ref[0]
    mu = jnp.sum(st[:, 0, :], axis=0, keepdims=True) * (1.0 / N)
    sq = jnp.sum(st[:, 1, :], axis=0, keepdims=True) * (1.0 / N)
    var = sq - mu * mu
    return (act_ref[0] - mu) * lax.rsqrt(var + 1e-5) * bng_ref[...] + bnb_ref[...]


def _tc_mid_body(act_ref, st_ref, dinv_ref, bng_ref, bnb_ref, w_ref, out_ref):
    h = _bn(act_ref, st_ref, bng_ref, bnb_ref)
    hw = jnp.dot(h, w_ref[...], preferred_element_type=jnp.float32)
    out_ref[0] = dinv_ref[0] * hw


def _tc_mid(act, st, dinv, bng, bnb, wn, dout, dnext):
    return pl.pallas_call(
        _tc_mid_body,
        grid=(B, RC),
        in_specs=[pl.BlockSpec((1, CHR, dout), lambda b, r: (b, r, 0)),
                  pl.BlockSpec((1, RC, 2, dout), lambda b, r: (b, 0, 0, 0)),
                  pl.BlockSpec((1, CHR, 1), lambda b, r: (b, r, 0)),
                  pl.BlockSpec((1, dout), lambda b, r: (0, 0)),
                  pl.BlockSpec((1, dout), lambda b, r: (0, 0)),
                  pl.BlockSpec((dout, dnext), lambda b, r: (0, 0))],
        out_specs=pl.BlockSpec((1, CHR, dnext), lambda b, r: (b, r, 0)),
        out_shape=jax.ShapeDtypeStruct((B, N, dnext), jnp.float32),
    )(act, st, dinv, bng, bnb, wn)


def _tc_last_body(act_ref, st_ref, bng_ref, bnb_ref, lng_ref, lnb_ref,
                  out_ref):
    h = _bn(act_ref, st_ref, bng_ref, bnb_ref)
    mu = jnp.mean(h, axis=-1, keepdims=True)
    xc = h - mu
    var = jnp.mean(xc * xc, axis=-1, keepdims=True)
    out_ref[0] = xc * lax.rsqrt(var + 1e-5) * lng_ref[...] + lnb_ref[...]


def _tc_last(act, st, bng, bnb, lng, lnb, dout):
    return pl.pallas_call(
        _tc_last_body,
        grid=(B, RC),
        in_specs=[pl.BlockSpec((1, CHR, dout), lambda b, r: (b, r, 0)),
                  pl.BlockSpec((1, RC, 2, dout), lambda b, r: (b, 0, 0, 0)),
                  pl.BlockSpec((1, dout), lambda b, r: (0, 0)),
                  pl.BlockSpec((1, dout), lambda b, r: (0, 0)),
                  pl.BlockSpec((1, dout), lambda b, r: (0, 0)),
                  pl.BlockSpec((1, dout), lambda b, r: (0, 0))],
        out_specs=pl.BlockSpec((1, CHR, dout), lambda b, r: (b, r, 0)),
        out_shape=jax.ShapeDtypeStruct((B, N, dout), jnp.float32),
    )(act, st, bng, bnb, lng, lnb)


def kernel(x, edge_index, params):
    xs = jnp.squeeze(x, -1)
    src = edge_index[:, 0, :] + (jnp.arange(B, dtype=jnp.int32) * N)[:, None]
    dst = edge_index[:, 1, :]
    pad = ((0, 0), (0, 0), (0, EPWP - EPW))
    srcp = jnp.pad(src.reshape(B, NW, EPW), pad)
    dpad = N + (jnp.arange(EPWP - EPW, dtype=jnp.int32) % (NPAD - N))
    dstp = jnp.concatenate(
        [dst.reshape(B, NW, EPW),
         jnp.broadcast_to(dpad, (B, NW, EPWP - EPW))], axis=2)
    def _pack(ch):
        nchk = EPWP // ch
        return jnp.stack([srcp.reshape(B, NW, nchk, ch),
                          dstp.reshape(B, NW, nchk, ch)], axis=3)
    packs = {c: _pack(c) for c in sorted({CH} | {_chunking(d)[0] for d in DOUTS})}
    ei128 = packs[CH]
    ones16 = jnp.ones((CH, 16), jnp.float32)
    z16 = jnp.zeros((NPAD, 16), jnp.float32)
    deg = _sc_deg(ei128, ones16, z16)
    dinv = _tc_dinv(deg)

    p = params
    vec = lambda v: v.reshape(1, -1)
    g = _tc_first(xs, dinv, p["W1"])
    for i in range(1, 6):
        dout = DOUTS[i - 1]
        zer = jnp.zeros((NPAD, dout), jnp.float32)
        ei = packs[_chunking(dout)[0]]
        s = _sc_agg(dout)(g.reshape(B * N, dout), ei, zer)
        act, st = _tc_act(s, g, dinv, vec(p[f"b{i}"]), dout)
        if i < 5:
            g = _tc_mid(act, st, dinv, vec(p[f"bn{i}_g"]), vec(p[f"bn{i}_b"]),
                        p[f"W{i+1}"], dout, DOUTS[i])
        else:
            out = _tc_last(act, st, vec(p["bn5_g"]), vec(p["bn5_b"]),
                           vec(p["ln_g"]), vec(p["ln_b"]), dout)
    return out.reshape(B, N * DOUTS[4])


# speedup vs baseline: 1.5358x; 1.5358x over previous
import functools

import jax
import jax.numpy as jnp
from jax import lax
from jax.experimental import pallas as pl
from jax.experimental.pallas import tpu as pltpu
from jax.experimental.pallas import tpu_sc as plsc

B, N, M, H, E = 4, 10000, 128, 32, 160000
NC, NS = 2, 16
NW = NC * NS
EPW = E // NW
CH = 128
NCHK = 40
EPWP = NCHK * CH
NBUF = 4
def _chunking(dout):
    if dout > 64:
        return 40, EPW // 40, 1
    return 128, EPWP // 128, 4
NPAD = 10112
RPT = NPAD // NS
DOUTS = [H, 2 * H, 4 * H, 4 * H, 4 * H]

_MESH = plsc.VectorSubcoreMesh(
    core_axis_name="c", subcore_axis_name="s", num_cores=NC, num_subcores=NS)
_SC_PARAMS = pltpu.CompilerParams(use_tc_tiling_on_sc=False)


def _sc_deg_body(ei_hbm, ones_hbm, zeros_hbm, out_hbm, slab, ones_v, acc,
                 sem):
    c = lax.axis_index("c")
    s = lax.axis_index("s")
    w = c * NS + s
    r0 = s * RPT
    pltpu.sync_copy(ones_hbm, ones_v)
    for b in range(B):
        pltpu.sync_copy(zeros_hbm.at[pl.ds(r0, RPT)], acc.at[pl.ds(r0, RPT)])
        plsc.subcore_barrier()
        pltpu.sync_copy(ei_hbm.at[b, w], slab)

        def fire(j, carry):
            pltpu.async_copy(ones_v, acc.at[slab.at[j, 1]], sem, add=True)
            return carry

        lax.fori_loop(0, NCHK, fire, 0)

        def drain(j, carry):
            pltpu.make_async_copy(ones_v, acc.at[slab.at[j, 1]], sem).wait()
            return carry

        lax.fori_loop(0, NCHK, drain, 0)
        plsc.subcore_barrier()
        pltpu.sync_copy(acc.at[pl.ds(r0, RPT)],
                        out_hbm.at[b, c, pl.ds(r0, RPT)])


_sc_deg = pl.kernel(
    _sc_deg_body,
    out_type=jax.ShapeDtypeStruct((B, NC, NPAD, 16), jnp.float32),
    mesh=_MESH,
    scratch_types=[
        pltpu.VMEM((NCHK, 2, CH), jnp.int32),
        pltpu.VMEM((CH, 16), jnp.float32),
        pltpu.VMEM_SHARED((NPAD, 16), jnp.float32),
        pltpu.SemaphoreType.DMA,
    ],
    compiler_params=_SC_PARAMS,
)


def _sc_agg_body(dout, g_hbm, ei_hbm, zeros_hbm, out_hbm, slab, rows, acc,
                 gsem, ssem):
    _, nchk, nbuf = _chunking(dout)
    lk = nbuf // 2
    c = lax.axis_index("c")
    s = lax.axis_index("s")
    w = c * NS + s
    r0 = s * RPT

    if nbuf == 1:
        def sync_step(j, carry):
            pltpu.sync_copy(g_hbm.at[slab.at[j, 0]], rows[0])
            pltpu.sync_copy(rows[0], acc.at[slab.at[j, 1]], add=True)
            return carry

    def step(j, k):
        k2 = (k + lk) % nbuf
        pltpu.make_async_copy(g_hbm.at[slab.at[j, 0]], rows[k], gsem[k]).wait()
        pltpu.async_copy(rows[k], acc.at[slab.at[j, 1]], ssem[k], add=True)

        @pl.when(j >= lk)
        def _():
            pltpu.make_async_copy(rows[k2], acc.at[slab.at[j - lk, 1]],
                                  ssem[k2]).wait()

        @pl.when(j < nchk - lk)
        def _():
            pltpu.async_copy(g_hbm.at[slab.at[j + lk, 0]], rows[k2], gsem[k2])

    for b in range(B):
        pltpu.sync_copy(zeros_hbm.at[pl.ds(r0, RPT)], acc.at[pl.ds(r0, RPT)])
        plsc.subcore_barrier()
        pltpu.sync_copy(ei_hbm.at[b, w], slab)
        if nbuf == 1:
            lax.fori_loop(0, nchk, sync_step, 0)
        else:
            for t in range(lk):
                pltpu.async_copy(g_hbm.at[slab.at[t, 0]], rows[t], gsem[t])

            def grp(m, carry):
                for k in range(nbuf):
                    step(nbuf * m + k, k)
                return carry

            lax.fori_loop(0, nchk // nbuf, grp, 0)
            for t in range(lk):
                j = nchk - lk + t
                pltpu.make_async_copy(rows[j % nbuf], acc.at[slab.at[j, 1]],
                                      ssem[j % nbuf]).wait()
        plsc.subcore_barrier()
        pltpu.sync_copy(acc.at[pl.ds(r0, RPT)],
                        out_hbm.at[b, c, pl.ds(r0, RPT)])


@functools.cache
def _sc_agg(dout):
    ch, nchk, nbuf = _chunking(dout)
    return pl.kernel(
        functools.partial(_sc_agg_body, dout),
        out_type=jax.ShapeDtypeStruct((B, NC, NPAD, dout), jnp.float32),
        mesh=_MESH,
        scratch_types=[
            pltpu.VMEM((nchk, 2, ch), jnp.int32),
            [pltpu.VMEM((ch, dout), jnp.float32)] * nbuf,
            pltpu.VMEM_SHARED((NPAD, dout), jnp.float32),
            [pltpu.SemaphoreType.DMA] * nbuf,
            [pltpu.SemaphoreType.DMA] * nbuf,
        ],
        compiler_params=_SC_PARAMS,
    )


RC = 5
CHR = N // RC


def _tc_dinv_body(deg_ref, out_ref):
    deg = deg_ref[0, 0, :, 0:1] + deg_ref[0, 1, :, 0:1] + 1.0
    out_ref[0] = lax.rsqrt(deg)


def _tc_dinv(deg):
    return pl.pallas_call(
        _tc_dinv_body,
        grid=(B,),
        in_specs=[pl.BlockSpec((1, NC, NPAD, 16), lambda b: (b, 0, 0, 0))],
        out_specs=pl.BlockSpec((1, NPAD, 1), lambda b: (b, 0, 0)),
        out_shape=jax.ShapeDtypeStruct((B, NPAD, 1), jnp.float32),
    )(deg)


def _tc_first_body(x_ref, dinv_ref, w_ref, g_ref):
    hw = jnp.dot(x_ref[0], w_ref[...], preferred_element_type=jnp.float32)
    g_ref[0] = dinv_ref[0] * hw


def _tc_first(xs, dinv, w1):
    return pl.pallas_call(
        _tc_first_body,
        grid=(B, RC),
        in_specs=[pl.BlockSpec((1, CHR, M), lambda b, r: (b, r, 0)),
                  pl.BlockSpec((1, CHR, 1), lambda b, r: (b, r, 0)),
                  pl.BlockSpec((M, H), lambda b, r: (0, 0))],
        out_specs=pl.BlockSpec((1, CHR, H), lambda b, r: (b, r, 0)),
        out_shape=jax.ShapeDtypeStruct((B, N, H), jnp.float32),
    )(xs, dinv, w1)


def _tc_act_body(s_ref, g_ref, dinv_ref, b_ref, act_ref, st_ref):
    tot = s_ref[0, 0] + s_ref[0, 1] + g_ref[0]
    pre = dinv_ref[0] * tot + b_ref[...]
    act = jnp.where(pre >= 0, pre, 0.01 * pre)
    act_ref[0] = act
    st_ref[0, 0, 0] = jnp.sum(act, axis=0)
    st_ref[0, 0, 1] = jnp.sum(act * act, axis=0)


def _tc_act(s, g, dinv, bi, dout):
    return pl.pallas_call(
        _tc_act_body,
        grid=(B, RC),
        in_specs=[pl.BlockSpec((1, NC, CHR, dout), lambda b, r: (b, 0, r, 0)),
                  pl.BlockSpec((1, CHR, dout), lambda b, r: (b, r, 0)),
                  pl.BlockSpec((1, CHR, 1), lambda b, r: (b, r, 0)),
                  pl.BlockSpec((1, dout), lambda b, r: (0, 0))],
        out_specs=[pl.BlockSpec((1, CHR, dout), lambda b, r: (b, r, 0)),
                   pl.BlockSpec((1, 1, 2, dout), lambda b, r: (b, r, 0, 0))],
        out_shape=[jax.ShapeDtypeStruct((B, N, dout), jnp.float32),
                   jax.ShapeDtypeStruct((B, RC, 2, dout), jnp.float32)],
    )(s, g, dinv, bi)


def _bn(act_ref, st_ref, bng_ref, bnb_ref):
    st = st_ref[0]
    mu = jnp.sum(st[:, 0, :], axis=0, keepdims=True) * (1.0 / N)
    sq = jnp.sum(st[:, 1, :], axis=0, keepdims=True) * (1.0 / N)
    var = sq - mu * mu
    return (act_ref[0] - mu) * lax.rsqrt(var + 1e-5) * bng_ref[...] + bnb_ref[...]


def _tc_mid_body(act_ref, st_ref, dinv_ref, bng_ref, bnb_ref, w_ref, out_ref):
    h = _bn(act_ref, st_ref, bng_ref, bnb_ref)
    hw = jnp.dot(h, w_ref[...], preferred_element_type=jnp.float32)
    out_ref[0] = dinv_ref[0] * hw


def _tc_mid(act, st, dinv, bng, bnb, wn, dout, dnext):
    return pl.pallas_call(
        _tc_mid_body,
        grid=(B, RC),
        in_specs=[pl.BlockSpec((1, CHR, dout), lambda b, r: (b, r, 0)),
                  pl.BlockSpec((1, RC, 2, dout), lambda b, r: (b, 0, 0, 0)),
                  pl.BlockSpec((1, CHR, 1), lambda b, r: (b, r, 0)),
                  pl.BlockSpec((1, dout), lambda b, r: (0, 0)),
                  pl.BlockSpec((1, dout), lambda b, r: (0, 0)),
                  pl.BlockSpec((dout, dnext), lambda b, r: (0, 0))],
        out_specs=pl.BlockSpec((1, CHR, dnext), lambda b, r: (b, r, 0)),
        out_shape=jax.ShapeDtypeStruct((B, N, dnext), jnp.float32),
    )(act, st, dinv, bng, bnb, wn)


def _tc_last_body(act_ref, st_ref, bng_ref, bnb_ref, lng_ref, lnb_ref,
                  out_ref):
    h = _bn(act_ref, st_ref, bng_ref, bnb_ref)
    mu = jnp.mean(h, axis=-1, keepdims=True)
    xc = h - mu
    var = jnp.mean(xc * xc, axis=-1, keepdims=True)
    out_ref[0] = xc * lax.rsqrt(var + 1e-5) * lng_ref[...] + lnb_ref[...]


def _tc_last(act, st, bng, bnb, lng, lnb, dout):
    return pl.pallas_call(
        _tc_last_body,
        grid=(B, RC),
        in_specs=[pl.BlockSpec((1, CHR, dout), lambda b, r: (b, r, 0)),
                  pl.BlockSpec((1, RC, 2, dout), lambda b, r: (b, 0, 0, 0)),
                  pl.BlockSpec((1, dout), lambda b, r: (0, 0)),
                  pl.BlockSpec((1, dout), lambda b, r: (0, 0)),
                  pl.BlockSpec((1, dout), lambda b, r: (0, 0)),
                  pl.BlockSpec((1, dout), lambda b, r: (0, 0))],
        out_specs=pl.BlockSpec((1, CHR, dout), lambda b, r: (b, r, 0)),
        out_shape=jax.ShapeDtypeStruct((B, N, dout), jnp.float32),
    )(act, st, bng, bnb, lng, lnb)


def kernel(x, edge_index, params):
    xs = jnp.squeeze(x, -1)
    src = edge_index[:, 0, :] + (jnp.arange(B, dtype=jnp.int32) * N)[:, None]
    dst = edge_index[:, 1, :]
    pad = ((0, 0), (0, 0), (0, EPWP - EPW))
    srcp = jnp.pad(src.reshape(B, NW, EPW), pad)
    dpad = N + (jnp.arange(EPWP - EPW, dtype=jnp.int32) % (NPAD - N))
    dstp = jnp.concatenate(
        [dst.reshape(B, NW, EPW),
         jnp.broadcast_to(dpad, (B, NW, EPWP - EPW))], axis=2)
    def _pack(ch):
        if EPW % ch == 0:
            nchk, s_, d_ = EPW // ch, src.reshape(B, NW, EPW), dst.reshape(B, NW, EPW)
        else:
            nchk, s_, d_ = EPWP // ch, srcp, dstp
        return jnp.stack([s_.reshape(B, NW, nchk, ch),
                          d_.reshape(B, NW, nchk, ch)], axis=3)
    packs = {c: _pack(c) for c in sorted({CH} | {_chunking(d)[0] for d in DOUTS})}
    ei128 = packs[CH]
    ones16 = jnp.ones((CH, 16), jnp.float32)
    z16 = jnp.zeros((NPAD, 16), jnp.float32)
    deg = _sc_deg(ei128, ones16, z16)
    dinv = _tc_dinv(deg)

    p = params
    vec = lambda v: v.reshape(1, -1)
    g = _tc_first(xs, dinv, p["W1"])
    for i in range(1, 6):
        dout = DOUTS[i - 1]
        zer = jnp.zeros((NPAD, dout), jnp.float32)
        ei = packs[_chunking(dout)[0]]
        s = _sc_agg(dout)(g.reshape(B * N, dout), ei, zer)
        act, st = _tc_act(s, g, dinv, vec(p[f"b{i}"]), dout)
        if i < 5:
            g = _tc_mid(act, st, dinv, vec(p[f"bn{i}_g"]), vec(p[f"bn{i}_b"]),
                        p[f"W{i+1}"], dout, DOUTS[i])
        else:
            out = _tc_last(act, st, vec(p["bn5_g"]), vec(p["bn5_b"]),
                           vec(p["ln_g"]), vec(p["ln_b"]), dout)
    return out.reshape(B, N * DOUTS[4])


# speedup vs baseline: 2.3508x; 1.5307x over previous
import functools

import jax
import jax.numpy as jnp
from jax import lax
from jax.experimental import pallas as pl
from jax.experimental.pallas import tpu as pltpu
from jax.experimental.pallas import tpu_sc as plsc

B, N, M, H, E = 4, 10000, 128, 32, 160000
NC, NS = 2, 16
NW = NC * NS
EPW = E // NW
CH = 128
NCHK = 40
EPWP = NCHK * CH
NBUF = 4
def _chunking(dout):
    return 40, EPW // 40, 5
NPAD = 10112
RPT = NPAD // NS
DOUTS = [H, 2 * H, 4 * H, 4 * H, 4 * H]

_MESH = plsc.VectorSubcoreMesh(
    core_axis_name="c", subcore_axis_name="s", num_cores=NC, num_subcores=NS)
_SC_PARAMS = pltpu.CompilerParams(use_tc_tiling_on_sc=False)


def _sc_deg_body(ei_hbm, ones_hbm, zeros_hbm, out_hbm, slab, ones_v, acc,
                 sem):
    c = lax.axis_index("c")
    s = lax.axis_index("s")
    w = c * NS + s
    r0 = s * RPT
    pltpu.sync_copy(ones_hbm, ones_v)
    for b in range(B):
        pltpu.sync_copy(zeros_hbm.at[pl.ds(r0, RPT)], acc.at[pl.ds(r0, RPT)])
        plsc.subcore_barrier()
        pltpu.sync_copy(ei_hbm.at[b, w], slab)

        def fire(j, carry):
            pltpu.async_copy(ones_v, acc.at[slab.at[j, 1]], sem, add=True)
            return carry

        lax.fori_loop(0, NCHK, fire, 0)

        def drain(j, carry):
            pltpu.make_async_copy(ones_v, acc.at[slab.at[j, 1]], sem).wait()
            return carry

        lax.fori_loop(0, NCHK, drain, 0)
        plsc.subcore_barrier()
        pltpu.sync_copy(acc.at[pl.ds(r0, RPT)],
                        out_hbm.at[b, c, pl.ds(r0, RPT)])


_sc_deg = pl.kernel(
    _sc_deg_body,
    out_type=jax.ShapeDtypeStruct((B, NC, NPAD, 16), jnp.float32),
    mesh=_MESH,
    scratch_types=[
        pltpu.VMEM((NCHK, 2, CH), jnp.int32),
        pltpu.VMEM((CH, 16), jnp.float32),
        pltpu.VMEM_SHARED((NPAD, 16), jnp.float32),
        pltpu.SemaphoreType.DMA,
    ],
    compiler_params=_SC_PARAMS,
)


def _sc_agg_body(dout, g_hbm, ei_hbm, zeros_hbm, out_hbm, slab, rows, acc,
                 gsem, ssem):
    _, nchk, nbuf = _chunking(dout)
    lk = nbuf // 2
    c = lax.axis_index("c")
    s = lax.axis_index("s")
    w = c * NS + s
    r0 = s * RPT

    if nbuf == 1:
        def sync_step(j, carry):
            pltpu.sync_copy(g_hbm.at[slab.at[j, 0]], rows[0])
            pltpu.sync_copy(rows[0], acc.at[slab.at[j, 1]], add=True)
            return carry

    def step(j, k):
        k2 = (k + lk) % nbuf
        k3 = (k - lk) % nbuf
        pltpu.make_async_copy(g_hbm.at[slab.at[j, 0]], rows[k], gsem[k]).wait()
        pltpu.async_copy(rows[k], acc.at[slab.at[j, 1]], ssem[k], add=True)

        @pl.when(j >= lk)
        def _():
            pltpu.make_async_copy(rows[k3], acc.at[slab.at[j - lk, 1]],
                                  ssem[k3]).wait()

        @pl.when(j < nchk - lk)
        def _():
            pltpu.async_copy(g_hbm.at[slab.at[j + lk, 0]], rows[k2], gsem[k2])

    for b in range(B):
        pltpu.sync_copy(zeros_hbm.at[pl.ds(r0, RPT)], acc.at[pl.ds(r0, RPT)])
        plsc.subcore_barrier()
        pltpu.sync_copy(ei_hbm.at[b, w], slab)
        if nbuf == 1:
            lax.fori_loop(0, nchk, sync_step, 0)
        else:
            for t in range(lk):
                pltpu.async_copy(g_hbm.at[slab.at[t, 0]], rows[t], gsem[t])

            def grp(m, carry):
                for k in range(nbuf):
                    step(nbuf * m + k, k)
                return carry

            lax.fori_loop(0, nchk // nbuf, grp, 0)
            for t in range(lk):
                j = nchk - lk + t
                pltpu.make_async_copy(rows[j % nbuf], acc.at[slab.at[j, 1]],
                                      ssem[j % nbuf]).wait()
        plsc.subcore_barrier()
        pltpu.sync_copy(acc.at[pl.ds(r0, RPT)],
                        out_hbm.at[b, c, pl.ds(r0, RPT)])


@functools.cache
def _sc_agg(dout):
    ch, nchk, nbuf = _chunking(dout)
    return pl.kernel(
        functools.partial(_sc_agg_body, dout),
        out_type=jax.ShapeDtypeStruct((B, NC, NPAD, dout), jnp.float32),
        mesh=_MESH,
        scratch_types=[
            pltpu.VMEM((nchk, 2, ch), jnp.int32),
            [pltpu.VMEM((ch, dout), jnp.float32)] * nbuf,
            pltpu.VMEM_SHARED((NPAD, dout), jnp.float32),
            [pltpu.SemaphoreType.DMA] * nbuf,
            [pltpu.SemaphoreType.DMA] * nbuf,
        ],
        compiler_params=_SC_PARAMS,
    )


RC = 5
CHR = N // RC


def _tc_dinv_body(deg_ref, out_ref):
    deg = deg_ref[0, 0, :, 0:1] + deg_ref[0, 1, :, 0:1] + 1.0
    out_ref[0] = lax.rsqrt(deg)


def _tc_dinv(deg):
    return pl.pallas_call(
        _tc_dinv_body,
        grid=(B,),
        in_specs=[pl.BlockSpec((1, NC, NPAD, 16), lambda b: (b, 0, 0, 0))],
        out_specs=pl.BlockSpec((1, NPAD, 1), lambda b: (b, 0, 0)),
        out_shape=jax.ShapeDtypeStruct((B, NPAD, 1), jnp.float32),
    )(deg)


def _tc_first_body(x_ref, dinv_ref, w_ref, g_ref):
    hw = jnp.dot(x_ref[0], w_ref[...], preferred_element_type=jnp.float32)
    g_ref[0] = dinv_ref[0] * hw


def _tc_first(xs, dinv, w1):
    return pl.pallas_call(
        _tc_first_body,
        grid=(B, RC),
        in_specs=[pl.BlockSpec((1, CHR, M), lambda b, r: (b, r, 0)),
                  pl.BlockSpec((1, CHR, 1), lambda b, r: (b, r, 0)),
                  pl.BlockSpec((M, H), lambda b, r: (0, 0))],
        out_specs=pl.BlockSpec((1, CHR, H), lambda b, r: (b, r, 0)),
        out_shape=jax.ShapeDtypeStruct((B, N, H), jnp.float32),
    )(xs, dinv, w1)


def _tc_act_body(s_ref, g_ref, dinv_ref, b_ref, act_ref, st_ref):
    tot = s_ref[0, 0] + s_ref[0, 1] + g_ref[0]
    pre = dinv_ref[0] * tot + b_ref[...]
    act = jnp.where(pre >= 0, pre, 0.01 * pre)
    act_ref[0] = act
    st_ref[0, 0, 0] = jnp.sum(act, axis=0)
    st_ref[0, 0, 1] = jnp.sum(act * act, axis=0)


def _tc_act(s, g, dinv, bi, dout):
    return pl.pallas_call(
        _tc_act_body,
        grid=(B, RC),
        in_specs=[pl.BlockSpec((1, NC, CHR, dout), lambda b, r: (b, 0, r, 0)),
                  pl.BlockSpec((1, CHR, dout), lambda b, r: (b, r, 0)),
                  pl.BlockSpec((1, CHR, 1), lambda b, r: (b, r, 0)),
                  pl.BlockSpec((1, dout), lambda b, r: (0, 0))],
        out_specs=[pl.BlockSpec((1, CHR, dout), lambda b, r: (b, r, 0)),
                   pl.BlockSpec((1, 1, 2, dout), lambda b, r: (b, r, 0, 0))],
        out_shape=[jax.ShapeDtypeStruct((B, N, dout), jnp.float32),
                   jax.ShapeDtypeStruct((B, RC, 2, dout), jnp.float32)],
    )(s, g, dinv, bi)


def _bn(act_ref, st_ref, bng_ref, bnb_ref):
    st = st_ref[0]
    mu = jnp.sum(st[:, 0, :], axis=0, keepdims=True) * (1.0 / N)
    sq = jnp.sum(st[:, 1, :], axis=0, keepdims=True) * (1.0 / N)
    var = sq - mu * mu
    return (act_ref[0] - mu) * lax.rsqrt(var + 1e-5) * bng_ref[...] + bnb_ref[...]


def _tc_mid_body(act_ref, st_ref, dinv_ref, bng_ref, bnb_ref, w_ref, out_ref):
    h = _bn(act_ref, st_ref, bng_ref, bnb_ref)
    hw = jnp.dot(h, w_ref[...], preferred_element_type=jnp.float32)
    out_ref[0] = dinv_ref[0] * hw


def _tc_mid(act, st, dinv, bng, bnb, wn, dout, dnext):
    return pl.pallas_call(
        _tc_mid_body,
        grid=(B, RC),
        in_specs=[pl.BlockSpec((1, CHR, dout), lambda b, r: (b, r, 0)),
                  pl.BlockSpec((1, RC, 2, dout), lambda b, r: (b, 0, 0, 0)),
                  pl.BlockSpec((1, CHR, 1), lambda b, r: (b, r, 0)),
                  pl.BlockSpec((1, dout), lambda b, r: (0, 0)),
                  pl.BlockSpec((1, dout), lambda b, r: (0, 0)),
                  pl.BlockSpec((dout, dnext), lambda b, r: (0, 0))],
        out_specs=pl.BlockSpec((1, CHR, dnext), lambda b, r: (b, r, 0)),
        out_shape=jax.ShapeDtypeStruct((B, N, dnext), jnp.float32),
    )(act, st, dinv, bng, bnb, wn)


def _tc_last_body(act_ref, st_ref, bng_ref, bnb_ref, lng_ref, lnb_ref,
                  out_ref):
    h = _bn(act_ref, st_ref, bng_ref, bnb_ref)
    mu = jnp.mean(h, axis=-1, keepdims=True)
    xc = h - mu
    var = jnp.mean(xc * xc, axis=-1, keepdims=True)
    out_ref[0] = xc * lax.rsqrt(var + 1e-5) * lng_ref[...] + lnb_ref[...]


def _tc_last(act, st, bng, bnb, lng, lnb, dout):
    return pl.pallas_call(
        _tc_last_body,
        grid=(B, RC),
        in_specs=[pl.BlockSpec((1, CHR, dout), lambda b, r: (b, r, 0)),
                  pl.BlockSpec((1, RC, 2, dout), lambda b, r: (b, 0, 0, 0)),
                  pl.BlockSpec((1, dout), lambda b, r: (0, 0)),
                  pl.BlockSpec((1, dout), lambda b, r: (0, 0)),
                  pl.BlockSpec((1, dout), lambda b, r: (0, 0)),
                  pl.BlockSpec((1, dout), lambda b, r: (0, 0))],
        out_specs=pl.BlockSpec((1, CHR, dout), lambda b, r: (b, r, 0)),
        out_shape=jax.ShapeDtypeStruct((B, N, dout), jnp.float32),
    )(act, st, bng, bnb, lng, lnb)


def kernel(x, edge_index, params):
    xs = jnp.squeeze(x, -1)
    src = edge_index[:, 0, :] + (jnp.arange(B, dtype=jnp.int32) * N)[:, None]
    dst = edge_index[:, 1, :]
    pad = ((0, 0), (0, 0), (0, EPWP - EPW))
    srcp = jnp.pad(src.reshape(B, NW, EPW), pad)
    dpad = N + (jnp.arange(EPWP - EPW, dtype=jnp.int32) % (NPAD - N))
    dstp = jnp.concatenate(
        [dst.reshape(B, NW, EPW),
         jnp.broadcast_to(dpad, (B, NW, EPWP - EPW))], axis=2)
    def _pack(ch):
        if EPW % ch == 0:
            nchk, s_, d_ = EPW // ch, src.reshape(B, NW, EPW), dst.reshape(B, NW, EPW)
        else:
            nchk, s_, d_ = EPWP // ch, srcp, dstp
        return jnp.stack([s_.reshape(B, NW, nchk, ch),
                          d_.reshape(B, NW, nchk, ch)], axis=3)
    packs = {c: _pack(c) for c in sorted({CH} | {_chunking(d)[0] for d in DOUTS})}
    ei128 = packs[CH]
    ones16 = jnp.ones((CH, 16), jnp.float32)
    z16 = jnp.zeros((NPAD, 16), jnp.float32)
    deg = _sc_deg(ei128, ones16, z16)
    dinv = _tc_dinv(deg)

    p = params
    vec = lambda v: v.reshape(1, -1)
    g = _tc_first(xs, dinv, p["W1"])
    for i in range(1, 6):
        dout = DOUTS[i - 1]
        zer = jnp.zeros((NPAD, dout), jnp.float32)
        ei = packs[_chunking(dout)[0]]
        s = _sc_agg(dout)(g.reshape(B * N, dout), ei, zer)
        act, st = _tc_act(s, g, dinv, vec(p[f"b{i}"]), dout)
        if i < 5:
            g = _tc_mid(act, st, dinv, vec(p[f"bn{i}_g"]), vec(p[f"bn{i}_b"]),
                        p[f"W{i+1}"], dout, DOUTS[i])
        else:
            out = _tc_last(act, st, vec(p["bn5_g"]), vec(p["bn5_b"]),
                           vec(p["ln_g"]), vec(p["ln_b"]), dout)
    return out.reshape(B, N * DOUTS[4])


# speedup vs baseline: 2.5489x; 1.0842x over previous
import functools

import jax
import jax.numpy as jnp
from jax import lax
from jax.experimental import pallas as pl
from jax.experimental.pallas import tpu as pltpu
from jax.experimental.pallas import tpu_sc as plsc

B, N, M, H, E = 4, 10000, 128, 32, 160000
NC, NS = 2, 16
NW = NC * NS
EPW = E // NW
CH = 40
NCHK = EPW // CH
NBUF = 5
LK = 2
NPAD = 10112
RPT = NPAD // NS
DOUTS = [H, 2 * H, 4 * H, 4 * H, 4 * H]

_MESH = plsc.VectorSubcoreMesh(
    core_axis_name="c", subcore_axis_name="s", num_cores=NC, num_subcores=NS)
_SC_PARAMS = pltpu.CompilerParams(use_tc_tiling_on_sc=False)


def _sc_deg_body(dst_hbm, ones_hbm, zeros_hbm, out_hbm, slab, ones_v, acc,
                 sem):
    c = lax.axis_index("c")
    s = lax.axis_index("s")
    w = c * NS + s
    r0 = s * RPT
    pltpu.sync_copy(ones_hbm, ones_v)
    for b in range(B):
        pltpu.sync_copy(zeros_hbm.at[pl.ds(r0, RPT)], acc.at[pl.ds(r0, RPT)])
        plsc.subcore_barrier()
        pltpu.sync_copy(dst_hbm.at[b, w], slab)

        def fire(j, carry):
            pltpu.async_copy(ones_v, acc.at[slab.at[j]], sem, add=True)
            return carry

        lax.fori_loop(0, NCHK, fire, 0)

        def drain(j, carry):
            pltpu.make_async_copy(ones_v, acc.at[slab.at[j]], sem).wait()
            return carry

        lax.fori_loop(0, NCHK, drain, 0)
        plsc.subcore_barrier()
        pltpu.sync_copy(acc.at[pl.ds(r0, RPT)],
                        out_hbm.at[b, c, pl.ds(r0, RPT)])


_sc_deg = pl.kernel(
    _sc_deg_body,
    out_type=jax.ShapeDtypeStruct((B, NC, NPAD, 16), jnp.float32),
    mesh=_MESH,
    scratch_types=[
        pltpu.VMEM((NCHK, CH), jnp.int32),
        pltpu.VMEM((CH, 16), jnp.float32),
        pltpu.VMEM_SHARED((NPAD, 16), jnp.float32),
        pltpu.SemaphoreType.DMA,
    ],
    compiler_params=_SC_PARAMS,
)


def _sc_agg_body(dout, g_hbm, src_hbm, dst_hbm, zeros_hbm, out_hbm,
                 slab_s, slab_d, rows, acc, gsem, ssem):
    c = lax.axis_index("c")
    s = lax.axis_index("s")
    w = c * NS + s
    r0 = s * RPT

    def step(j, k):
        k2 = (k + LK) % NBUF
        k3 = (k - LK) % NBUF
        pltpu.make_async_copy(g_hbm.at[slab_s.at[j]], rows[k], gsem[k]).wait()
        pltpu.async_copy(rows[k], acc.at[slab_d.at[j]], ssem[k], add=True)

        @pl.when(j >= LK)
        def _():
            pltpu.make_async_copy(rows[k3], acc.at[slab_d.at[j - LK]],
                                  ssem[k3]).wait()

        @pl.when(j < NCHK - LK)
        def _():
            pltpu.async_copy(g_hbm.at[slab_s.at[j + LK]], rows[k2], gsem[k2])

    for b in range(B):
        pltpu.sync_copy(zeros_hbm.at[pl.ds(r0, RPT)], acc.at[pl.ds(r0, RPT)])
        plsc.subcore_barrier()
        pltpu.sync_copy(src_hbm.at[b, w], slab_s)
        pltpu.sync_copy(dst_hbm.at[b, w], slab_d)
        for t in range(LK):
            pltpu.async_copy(g_hbm.at[slab_s.at[t]], rows[t], gsem[t])

        def grp(m, carry):
            for k in range(NBUF):
                step(NBUF * m + k, k)
            return carry

        lax.fori_loop(0, NCHK // NBUF, grp, 0)
        for t in range(LK):
            j = NCHK - LK + t
            pltpu.make_async_copy(rows[j % NBUF], acc.at[slab_d.at[j]],
                                  ssem[j % NBUF]).wait()
        plsc.subcore_barrier()
        pltpu.sync_copy(acc.at[pl.ds(r0, RPT)],
                        out_hbm.at[b, c, pl.ds(r0, RPT)])


@functools.cache
def _sc_agg(dout):
    return pl.kernel(
        functools.partial(_sc_agg_body, dout),
        out_type=jax.ShapeDtypeStruct((B, NC, NPAD, dout), jnp.float32),
        mesh=_MESH,
        scratch_types=[
            pltpu.VMEM((NCHK, CH), jnp.int32),
            pltpu.VMEM((NCHK, CH), jnp.int32),
            [pltpu.VMEM((CH, dout), jnp.float32)] * NBUF,
            pltpu.VMEM_SHARED((NPAD, dout), jnp.float32),
            [pltpu.SemaphoreType.DMA] * NBUF,
            [pltpu.SemaphoreType.DMA] * NBUF,
        ],
        compiler_params=_SC_PARAMS,
    )


RC = 5
CHR = N // RC


def _tc_dinv_body(deg_ref, out_ref):
    deg = deg_ref[0, 0, :, 0:1] + deg_ref[0, 1, :, 0:1] + 1.0
    out_ref[0] = lax.rsqrt(deg)


def _tc_dinv(deg):
    return pl.pallas_call(
        _tc_dinv_body,
        grid=(B,),
        in_specs=[pl.BlockSpec((1, NC, NPAD, 16), lambda b: (b, 0, 0, 0))],
        out_specs=pl.BlockSpec((1, NPAD, 1), lambda b: (b, 0, 0)),
        out_shape=jax.ShapeDtypeStruct((B, NPAD, 1), jnp.float32),
    )(deg)


def _tc_first_body(x_ref, dinv_ref, w_ref, g_ref):
    hw = jnp.dot(x_ref[0], w_ref[...], preferred_element_type=jnp.float32)
    g_ref[0] = dinv_ref[0] * hw


def _tc_first(x, dinv, w1):
    return pl.pallas_call(
        _tc_first_body,
        grid=(B, RC),
        in_specs=[pl.BlockSpec((1, CHR, M), lambda b, r: (b, r, 0)),
                  pl.BlockSpec((1, CHR, 1), lambda b, r: (b, r, 0)),
                  pl.BlockSpec((M, H), lambda b, r: (0, 0))],
        out_specs=pl.BlockSpec((1, CHR, H), lambda b, r: (b, r, 0)),
        out_shape=jax.ShapeDtypeStruct((B, N, H), jnp.float32),
    )(x, dinv, w1)


def _tc_act_body(s_ref, g_ref, dinv_ref, b_ref, act_ref, st_ref):
    tot = s_ref[0, 0] + s_ref[0, 1] + g_ref[0]
    pre = dinv_ref[0] * tot + b_ref[...]
    act = jnp.where(pre >= 0, pre, 0.01 * pre)
    act_ref[0] = act
    st_ref[0, 0, 0] = jnp.sum(act, axis=0)
    st_ref[0, 0, 1] = jnp.sum(act * act, axis=0)


def _tc_act(s, g, dinv, bi, dout):
    return pl.pallas_call(
        _tc_act_body,
        grid=(B, RC),
        in_specs=[pl.BlockSpec((1, NC, CHR, dout), lambda b, r: (b, 0, r, 0)),
                  pl.BlockSpec((1, CHR, dout), lambda b, r: (b, r, 0)),
                  pl.BlockSpec((1, CHR, 1), lambda b, r: (b, r, 0)),
                  pl.BlockSpec((1, dout), lambda b, r: (0, 0))],
        out_specs=[pl.BlockSpec((1, CHR, dout), lambda b, r: (b, r, 0)),
                   pl.BlockSpec((1, 1, 2, dout), lambda b, r: (b, r, 0, 0))],
        out_shape=[jax.ShapeDtypeStruct((B, N, dout), jnp.float32),
                   jax.ShapeDtypeStruct((B, RC, 2, dout), jnp.float32)],
    )(s, g, dinv, bi)


def _bn(act_ref, st_ref, bng_ref, bnb_ref):
    st = st_ref[0]
    mu = jnp.sum(st[:, 0, :], axis=0, keepdims=True) * (1.0 / N)
    sq = jnp.sum(st[:, 1, :], axis=0, keepdims=True) * (1.0 / N)
    var = sq - mu * mu
    return (act_ref[0] - mu) * lax.rsqrt(var + 1e-5) * bng_ref[...] + bnb_ref[...]


def _tc_mid_body(act_ref, st_ref, dinv_ref, bng_ref, bnb_ref, w_ref, out_ref):
    h = _bn(act_ref, st_ref, bng_ref, bnb_ref)
    hw = jnp.dot(h, w_ref[...], preferred_element_type=jnp.float32)
    out_ref[0] = dinv_ref[0] * hw


def _tc_mid(act, st, dinv, bng, bnb, wn, dout, dnext):
    return pl.pallas_call(
        _tc_mid_body,
        grid=(B, RC),
        in_specs=[pl.BlockSpec((1, CHR, dout), lambda b, r: (b, r, 0)),
                  pl.BlockSpec((1, RC, 2, dout), lambda b, r: (b, 0, 0, 0)),
                  pl.BlockSpec((1, CHR, 1), lambda b, r: (b, r, 0)),
                  pl.BlockSpec((1, dout), lambda b, r: (0, 0)),
                  pl.BlockSpec((1, dout), lambda b, r: (0, 0)),
                  pl.BlockSpec((dout, dnext), lambda b, r: (0, 0))],
        out_specs=pl.BlockSpec((1, CHR, dnext), lambda b, r: (b, r, 0)),
        out_shape=jax.ShapeDtypeStruct((B, N, dnext), jnp.float32),
    )(act, st, dinv, bng, bnb, wn)


def _tc_last_body(act_ref, st_ref, bng_ref, bnb_ref, lng_ref, lnb_ref,
                  out_ref):
    h = _bn(act_ref, st_ref, bng_ref, bnb_ref)
    mu = jnp.mean(h, axis=-1, keepdims=True)
    xc = h - mu
    var = jnp.mean(xc * xc, axis=-1, keepdims=True)
    out_ref[0] = xc * lax.rsqrt(var + 1e-5) * lng_ref[...] + lnb_ref[...]


def _tc_last(act, st, bng, bnb, lng, lnb, dout):
    return pl.pallas_call(
        _tc_last_body,
        grid=(B, RC),
        in_specs=[pl.BlockSpec((1, CHR, dout), lambda b, r: (b, r, 0)),
                  pl.BlockSpec((1, RC, 2, dout), lambda b, r: (b, 0, 0, 0)),
                  pl.BlockSpec((1, dout), lambda b, r: (0, 0)),
                  pl.BlockSpec((1, dout), lambda b, r: (0, 0)),
                  pl.BlockSpec((1, dout), lambda b, r: (0, 0)),
                  pl.BlockSpec((1, dout), lambda b, r: (0, 0))],
        out_specs=pl.BlockSpec((1, CHR, dout), lambda b, r: (b, r, 0)),
        out_shape=jax.ShapeDtypeStruct((B, N, dout), jnp.float32),
    )(act, st, bng, bnb, lng, lnb)


def kernel(x, edge_index, params):
    xs = jnp.squeeze(x, -1)
    src = edge_index[:, 0, :] + (jnp.arange(B, dtype=jnp.int32) * N)[:, None]
    srcp = src.reshape(B, NW, NCHK, CH)
    dstp = edge_index[:, 1, :].reshape(B, NW, NCHK, CH)
    ones16 = jnp.ones((CH, 16), jnp.float32)
    z16 = jnp.zeros((NPAD, 16), jnp.float32)
    deg = _sc_deg(dstp, ones16, z16)
    dinv = _tc_dinv(deg)

    p = params
    vec = lambda v: v.reshape(1, -1)
    g = _tc_first(xs, dinv, p["W1"])
    for i in range(1, 6):
        dout = DOUTS[i - 1]
        zer = jnp.zeros((NPAD, dout), jnp.float32)
        s = _sc_agg(dout)(g.reshape(B * N, dout), srcp, dstp, zer)
        act, st = _tc_act(s, g, dinv, vec(p[f"b{i}"]), dout)
        if i < 5:
            g = _tc_mid(act, st, dinv, vec(p[f"bn{i}_g"]), vec(p[f"bn{i}_b"]),
                        p[f"W{i+1}"], dout, DOUTS[i])
        else:
            out = _tc_last(act, st, vec(p["bn5_g"]), vec(p["bn5_b"]),
                           vec(p["ln_g"]), vec(p["ln_b"]), dout)
    return out.reshape(B, N * DOUTS[4])


# speedup vs baseline: 2.6055x; 1.0222x over previous
import functools

import jax
import jax.numpy as jnp
from jax import lax
from jax.experimental import pallas as pl
from jax.experimental.pallas import tpu as pltpu
from jax.experimental.pallas import tpu_sc as plsc

B, N, M, H, E = 4, 10000, 128, 32, 160000
NC, NS = 2, 16
NW = NC * NS
EPW = E // NW
CH = 40
NCHK = EPW // CH
NBUF = 5
LK = 2
NPAD = 10112
RPT = NPAD // NS
DOUTS = [H, 2 * H, 4 * H, 4 * H, 4 * H]

_MESH = plsc.VectorSubcoreMesh(
    core_axis_name="c", subcore_axis_name="s", num_cores=NC, num_subcores=NS)
_SC_PARAMS = pltpu.CompilerParams(use_tc_tiling_on_sc=False)


def _sc_deg_body(dst_hbm, ones_hbm, zeros_hbm, out_hbm, slab, ones_v, acc,
                 sem):
    c = lax.axis_index("c")
    s = lax.axis_index("s")
    w = c * NS + s
    r0 = s * RPT
    pltpu.sync_copy(ones_hbm, ones_v)
    for b in range(B):
        pltpu.sync_copy(zeros_hbm.at[pl.ds(r0, RPT)], acc.at[pl.ds(r0, RPT)])
        plsc.subcore_barrier()
        pltpu.sync_copy(dst_hbm.at[b, w], slab)

        def fire(j, carry):
            pltpu.async_copy(ones_v, acc.at[slab.at[j]], sem, add=True)
            return carry

        lax.fori_loop(0, NCHK, fire, 0)

        def drain(j, carry):
            pltpu.make_async_copy(ones_v, acc.at[slab.at[j]], sem).wait()
            return carry

        lax.fori_loop(0, NCHK, drain, 0)
        plsc.subcore_barrier()
        pltpu.sync_copy(acc.at[pl.ds(r0, RPT)],
                        out_hbm.at[b, c, pl.ds(r0, RPT)])


_sc_deg = pl.kernel(
    _sc_deg_body,
    out_type=jax.ShapeDtypeStruct((B, NC, NPAD, 16), jnp.float32),
    mesh=_MESH,
    scratch_types=[
        pltpu.VMEM((NCHK, CH), jnp.int32),
        pltpu.VMEM((CH, 16), jnp.float32),
        pltpu.VMEM_SHARED((NPAD, 16), jnp.float32),
        pltpu.SemaphoreType.DMA,
    ],
    compiler_params=_SC_PARAMS,
)


def _sc_agg_body(dout, g_hbm, src_hbm, dst_hbm, zeros_hbm, out_hbm,
                 slab_s, slab_d, rows, acc, gsem, ssem):
    c = lax.axis_index("c")
    s = lax.axis_index("s")
    w = c * NS + s
    r0 = s * RPT

    def step(j, k):
        k2 = (k + LK) % NBUF
        k3 = (k - LK) % NBUF
        pltpu.make_async_copy(g_hbm.at[slab_s.at[j]], rows[k], gsem[k]).wait()
        pltpu.async_copy(rows[k], acc.at[slab_d.at[j]], ssem[k], add=True)

        @pl.when(j >= LK)
        def _():
            pltpu.make_async_copy(rows[k3], acc.at[slab_d.at[j - LK]],
                                  ssem[k3]).wait()

        @pl.when(j < NCHK - LK)
        def _():
            pltpu.async_copy(g_hbm.at[slab_s.at[j + LK]], rows[k2], gsem[k2])

    for b in range(B):
        pltpu.sync_copy(zeros_hbm.at[pl.ds(r0, RPT)], acc.at[pl.ds(r0, RPT)])
        plsc.subcore_barrier()
        pltpu.sync_copy(src_hbm.at[b, w], slab_s)
        pltpu.sync_copy(dst_hbm.at[b, w], slab_d)
        for t in range(LK):
            pltpu.async_copy(g_hbm.at[slab_s.at[t]], rows[t], gsem[t])

        def grp(m, carry):
            for k in range(NBUF):
                step(NBUF * m + k, k)
            return carry

        lax.fori_loop(0, NCHK // NBUF, grp, 0)
        for t in range(LK):
            j = NCHK - LK + t
            pltpu.make_async_copy(rows[j % NBUF], acc.at[slab_d.at[j]],
                                  ssem[j % NBUF]).wait()
        plsc.subcore_barrier()
        pltpu.sync_copy(acc.at[pl.ds(r0, RPT)],
                        out_hbm.at[b, c, pl.ds(r0, RPT)])


@functools.cache
def _sc_agg(dout):
    return pl.kernel(
        functools.partial(_sc_agg_body, dout),
        out_type=jax.ShapeDtypeStruct((B, NC, NPAD, dout), jnp.float32),
        mesh=_MESH,
        scratch_types=[
            pltpu.VMEM((NCHK, CH), jnp.int32),
            pltpu.VMEM((NCHK, CH), jnp.int32),
            [pltpu.VMEM((CH, dout), jnp.float32)] * NBUF,
            pltpu.VMEM_SHARED((NPAD, dout), jnp.float32),
            [pltpu.SemaphoreType.DMA] * NBUF,
            [pltpu.SemaphoreType.DMA] * NBUF,
        ],
        compiler_params=_SC_PARAMS,
    )


RC = 2
CHR = N // RC


def _tc_dinv_body(deg_ref, out_ref):
    deg = deg_ref[0, 0, :, 0:1] + deg_ref[0, 1, :, 0:1] + 1.0
    out_ref[0] = lax.rsqrt(deg)


def _tc_dinv(deg):
    return pl.pallas_call(
        _tc_dinv_body,
        grid=(B,),
        in_specs=[pl.BlockSpec((1, NC, NPAD, 16), lambda b: (b, 0, 0, 0))],
        out_specs=pl.BlockSpec((1, NPAD, 1), lambda b: (b, 0, 0)),
        out_shape=jax.ShapeDtypeStruct((B, NPAD, 1), jnp.float32),
    )(deg)


def _tc_first_body(x_ref, dinv_ref, w_ref, g_ref):
    hw = jnp.dot(x_ref[0], w_ref[...], preferred_element_type=jnp.float32)
    g_ref[0] = dinv_ref[0] * hw


def _tc_first(x, dinv, w1):
    return pl.pallas_call(
        _tc_first_body,
        grid=(B, RC),
        in_specs=[pl.BlockSpec((1, CHR, M), lambda b, r: (b, r, 0)),
                  pl.BlockSpec((1, CHR, 1), lambda b, r: (b, r, 0)),
                  pl.BlockSpec((M, H), lambda b, r: (0, 0))],
        out_specs=pl.BlockSpec((1, CHR, H), lambda b, r: (b, r, 0)),
        out_shape=jax.ShapeDtypeStruct((B, N, H), jnp.float32),
    )(x, dinv, w1)


def _tc_act_body(s_ref, g_ref, dinv_ref, b_ref, act_ref, st_ref):
    tot = s_ref[0, 0] + s_ref[0, 1] + g_ref[0]
    pre = dinv_ref[0] * tot + b_ref[...]
    act = jnp.where(pre >= 0, pre, 0.01 * pre)
    act_ref[0] = act
    st_ref[0, 0, 0] = jnp.sum(act, axis=0)
    st_ref[0, 0, 1] = jnp.sum(act * act, axis=0)


def _tc_act(s, g, dinv, bi, dout):
    return pl.pallas_call(
        _tc_act_body,
        grid=(B, RC),
        in_specs=[pl.BlockSpec((1, NC, CHR, dout), lambda b, r: (b, 0, r, 0)),
                  pl.BlockSpec((1, CHR, dout), lambda b, r: (b, r, 0)),
                  pl.BlockSpec((1, CHR, 1), lambda b, r: (b, r, 0)),
                  pl.BlockSpec((1, dout), lambda b, r: (0, 0))],
        out_specs=[pl.BlockSpec((1, CHR, dout), lambda b, r: (b, r, 0)),
                   pl.BlockSpec((1, 1, 2, dout), lambda b, r: (b, r, 0, 0))],
        out_shape=[jax.ShapeDtypeStruct((B, N, dout), jnp.float32),
                   jax.ShapeDtypeStruct((B, RC, 2, dout), jnp.float32)],
    )(s, g, dinv, bi)


def _bn(act_ref, st_ref, bng_ref, bnb_ref):
    st = st_ref[0]
    mu = jnp.sum(st[:, 0, :], axis=0, keepdims=True) * (1.0 / N)
    sq = jnp.sum(st[:, 1, :], axis=0, keepdims=True) * (1.0 / N)
    var = sq - mu * mu
    return (act_ref[0] - mu) * lax.rsqrt(var + 1e-5) * bng_ref[...] + bnb_ref[...]


def _tc_mid_body(act_ref, st_ref, dinv_ref, bng_ref, bnb_ref, w_ref, out_ref):
    h = _bn(act_ref, st_ref, bng_ref, bnb_ref)
    hw = jnp.dot(h, w_ref[...], preferred_element_type=jnp.float32)
    out_ref[0] = dinv_ref[0] * hw


def _tc_mid(act, st, dinv, bng, bnb, wn, dout, dnext):
    return pl.pallas_call(
        _tc_mid_body,
        grid=(B, RC),
        in_specs=[pl.BlockSpec((1, CHR, dout), lambda b, r: (b, r, 0)),
                  pl.BlockSpec((1, RC, 2, dout), lambda b, r: (b, 0, 0, 0)),
                  pl.BlockSpec((1, CHR, 1), lambda b, r: (b, r, 0)),
                  pl.BlockSpec((1, dout), lambda b, r: (0, 0)),
                  pl.BlockSpec((1, dout), lambda b, r: (0, 0)),
                  pl.BlockSpec((dout, dnext), lambda b, r: (0, 0))],
        out_specs=pl.BlockSpec((1, CHR, dnext), lambda b, r: (b, r, 0)),
        out_shape=jax.ShapeDtypeStruct((B, N, dnext), jnp.float32),
    )(act, st, dinv, bng, bnb, wn)


def _tc_last_body(act_ref, st_ref, bng_ref, bnb_ref, lng_ref, lnb_ref,
                  out_ref):
    h = _bn(act_ref, st_ref, bng_ref, bnb_ref)
    mu = jnp.mean(h, axis=-1, keepdims=True)
    xc = h - mu
    var = jnp.mean(xc * xc, axis=-1, keepdims=True)
    out_ref[0] = xc * lax.rsqrt(var + 1e-5) * lng_ref[...] + lnb_ref[...]


def _tc_last(act, st, bng, bnb, lng, lnb, dout):
    return pl.pallas_call(
        _tc_last_body,
        grid=(B, RC),
        in_specs=[pl.BlockSpec((1, CHR, dout), lambda b, r: (b, r, 0)),
                  pl.BlockSpec((1, RC, 2, dout), lambda b, r: (b, 0, 0, 0)),
                  pl.BlockSpec((1, dout), lambda b, r: (0, 0)),
                  pl.BlockSpec((1, dout), lambda b, r: (0, 0)),
                  pl.BlockSpec((1, dout), lambda b, r: (0, 0)),
                  pl.BlockSpec((1, dout), lambda b, r: (0, 0))],
        out_specs=pl.BlockSpec((1, CHR, dout), lambda b, r: (b, r, 0)),
        out_shape=jax.ShapeDtypeStruct((B, N, dout), jnp.float32),
    )(act, st, bng, bnb, lng, lnb)


def kernel(x, edge_index, params):
    xs = jnp.squeeze(x, -1)
    src = edge_index[:, 0, :] + (jnp.arange(B, dtype=jnp.int32) * N)[:, None]
    srcp = src.reshape(B, NW, NCHK, CH)
    dstp = edge_index[:, 1, :].reshape(B, NW, NCHK, CH)
    ones16 = jnp.ones((CH, 16), jnp.float32)
    z16 = jnp.zeros((NPAD, 16), jnp.float32)
    deg = _sc_deg(dstp, ones16, z16)
    dinv = _tc_dinv(deg)

    p = params
    vec = lambda v: v.reshape(1, -1)
    g = _tc_first(xs, dinv, p["W1"])
    for i in range(1, 6):
        dout = DOUTS[i - 1]
        zer = jnp.zeros((NPAD, dout), jnp.float32)
        s = _sc_agg(dout)(g.reshape(B * N, dout), srcp, dstp, zer)
        act, st = _tc_act(s, g, dinv, vec(p[f"b{i}"]), dout)
        if i < 5:
            g = _tc_mid(act, st, dinv, vec(p[f"bn{i}_g"]), vec(p[f"bn{i}_b"]),
                        p[f"W{i+1}"], dout, DOUTS[i])
        else:
            out = _tc_last(act, st, vec(p["bn5_g"]), vec(p["bn5_b"]),
                           vec(p["ln_g"]), vec(p["ln_b"]), dout)
    return out.reshape(B, N * DOUTS[4])


# speedup vs baseline: 2.6090x; 1.0013x over previous
import functools

import jax
import jax.numpy as jnp
from jax import lax
from jax.experimental import pallas as pl
from jax.experimental.pallas import tpu as pltpu
from jax.experimental.pallas import tpu_sc as plsc

B, N, M, H, E = 4, 10000, 128, 32, 160000
NC, NS = 2, 16
NW = NC * NS
EPW = E // NW
CH = 40
NCHK = EPW // CH
NBUF = 5
LK = 2
NPAD = 10112
RPT = NPAD // NS
DOUTS = [H, 2 * H, 4 * H, 4 * H, 4 * H]

_MESH = plsc.VectorSubcoreMesh(
    core_axis_name="c", subcore_axis_name="s", num_cores=NC, num_subcores=NS)
_SC_PARAMS = pltpu.CompilerParams(use_tc_tiling_on_sc=False)


def _sc_deg_body(dst_hbm, ones_hbm, zeros_hbm, out_hbm, slab, ones_v, acc,
                 sem):
    c = lax.axis_index("c")
    s = lax.axis_index("s")
    w = c * NS + s
    r0 = s * RPT
    pltpu.sync_copy(ones_hbm, ones_v)
    for b in range(B):
        pltpu.sync_copy(zeros_hbm.at[pl.ds(r0, RPT)], acc.at[pl.ds(r0, RPT)])
        plsc.subcore_barrier()
        pltpu.sync_copy(dst_hbm.at[b, w], slab)

        def fire(j, carry):
            pltpu.async_copy(ones_v, acc.at[slab.at[j]], sem, add=True)
            return carry

        lax.fori_loop(0, NCHK, fire, 0)

        def drain(j, carry):
            pltpu.make_async_copy(ones_v, acc.at[slab.at[j]], sem).wait()
            return carry

        lax.fori_loop(0, NCHK, drain, 0)
        plsc.subcore_barrier()
        pltpu.sync_copy(acc.at[pl.ds(r0, RPT)],
                        out_hbm.at[b, c, pl.ds(r0, RPT)])


_sc_deg = pl.kernel(
    _sc_deg_body,
    out_type=jax.ShapeDtypeStruct((B, NC, NPAD, 16), jnp.float32),
    mesh=_MESH,
    scratch_types=[
        pltpu.VMEM((NCHK, CH), jnp.int32),
        pltpu.VMEM((CH, 16), jnp.float32),
        pltpu.VMEM_SHARED((NPAD, 16), jnp.float32),
        pltpu.SemaphoreType.DMA,
    ],
    compiler_params=_SC_PARAMS,
)


def _sc_agg_body(dout, g_hbm, src_hbm, dst_hbm, zeros_hbm, out_hbm,
                 slab_s, slab_d, rows, acc, gsem, ssem):
    c = lax.axis_index("c")
    s = lax.axis_index("s")
    w = c * NS + s
    r0 = s * RPT

    def step(j, k):
        k2 = (k + LK) % NBUF
        k3 = (k - LK) % NBUF
        pltpu.make_async_copy(g_hbm.at[slab_s.at[j]], rows[k], gsem[k]).wait()
        pltpu.async_copy(rows[k], acc.at[slab_d.at[j]], ssem[k], add=True)

        @pl.when(j >= LK)
        def _():
            pltpu.make_async_copy(rows[k3], acc.at[slab_d.at[j - LK]],
                                  ssem[k3]).wait()

        @pl.when(j < NCHK - LK)
        def _():
            pltpu.async_copy(g_hbm.at[slab_s.at[j + LK]], rows[k2], gsem[k2])

    for b in range(B):
        pltpu.sync_copy(zeros_hbm.at[pl.ds(r0, RPT)], acc.at[pl.ds(r0, RPT)])
        plsc.subcore_barrier()
        pltpu.sync_copy(src_hbm.at[b, w], slab_s)
        pltpu.sync_copy(dst_hbm.at[b, w], slab_d)
        for t in range(LK):
            pltpu.async_copy(g_hbm.at[slab_s.at[t]], rows[t], gsem[t])

        def grp(m, carry):
            for k in range(NBUF):
                step(NBUF * m + k, k)
            return carry

        lax.fori_loop(0, NCHK // NBUF, grp, 0)
        for t in range(LK):
            j = NCHK - LK + t
            pltpu.make_async_copy(rows[j % NBUF], acc.at[slab_d.at[j]],
                                  ssem[j % NBUF]).wait()
        plsc.subcore_barrier()
        pltpu.sync_copy(acc.at[pl.ds(r0, RPT)],
                        out_hbm.at[b, c, pl.ds(r0, RPT)])


@functools.cache
def _sc_agg(dout):
    return pl.kernel(
        functools.partial(_sc_agg_body, dout),
        out_type=jax.ShapeDtypeStruct((B, NC, NPAD, dout), jnp.float32),
        mesh=_MESH,
        scratch_types=[
            pltpu.VMEM((NCHK, CH), jnp.int32),
            pltpu.VMEM((NCHK, CH), jnp.int32),
            [pltpu.VMEM((CH, dout), jnp.float32)] * NBUF,
            pltpu.VMEM_SHARED((NPAD, dout), jnp.float32),
            [pltpu.SemaphoreType.DMA] * NBUF,
            [pltpu.SemaphoreType.DMA] * NBUF,
        ],
        compiler_params=_SC_PARAMS,
    )


RC = 1
CHR = N // RC


def _tc_dinv_body(deg_ref, out_ref):
    deg = deg_ref[0, 0, :, 0:1] + deg_ref[0, 1, :, 0:1] + 1.0
    out_ref[0] = lax.rsqrt(deg)


def _tc_dinv(deg):
    return pl.pallas_call(
        _tc_dinv_body,
        grid=(B,),
        in_specs=[pl.BlockSpec((1, NC, NPAD, 16), lambda b: (b, 0, 0, 0))],
        out_specs=pl.BlockSpec((1, NPAD, 1), lambda b: (b, 0, 0)),
        out_shape=jax.ShapeDtypeStruct((B, NPAD, 1), jnp.float32),
    )(deg)


def _tc_first_body(x_ref, dinv_ref, w_ref, g_ref):
    hw = jnp.dot(x_ref[0], w_ref[...], preferred_element_type=jnp.float32)
    g_ref[0] = dinv_ref[0] * hw


def _tc_first(x, dinv, w1):
    return pl.pallas_call(
        _tc_first_body,
        grid=(B, RC),
        in_specs=[pl.BlockSpec((1, CHR, M), lambda b, r: (b, r, 0)),
                  pl.BlockSpec((1, CHR, 1), lambda b, r: (b, r, 0)),
                  pl.BlockSpec((M, H), lambda b, r: (0, 0))],
        out_specs=pl.BlockSpec((1, CHR, H), lambda b, r: (b, r, 0)),
        out_shape=jax.ShapeDtypeStruct((B, N, H), jnp.float32),
    )(x, dinv, w1)


def _tc_act_body(s_ref, g_ref, dinv_ref, b_ref, act_ref, st_ref):
    tot = s_ref[0, 0] + s_ref[0, 1] + g_ref[0]
    pre = dinv_ref[0] * tot + b_ref[...]
    act = jnp.where(pre >= 0, pre, 0.01 * pre)
    act_ref[0] = act
    st_ref[0, 0, 0] = jnp.sum(act, axis=0)
    st_ref[0, 0, 1] = jnp.sum(act * act, axis=0)


def _tc_act(s, g, dinv, bi, dout):
    return pl.pallas_call(
        _tc_act_body,
        grid=(B, RC),
        in_specs=[pl.BlockSpec((1, NC, CHR, dout), lambda b, r: (b, 0, r, 0)),
                  pl.BlockSpec((1, CHR, dout), lambda b, r: (b, r, 0)),
                  pl.BlockSpec((1, CHR, 1), lambda b, r: (b, r, 0)),
                  pl.BlockSpec((1, dout), lambda b, r: (0, 0))],
        out_specs=[pl.BlockSpec((1, CHR, dout), lambda b, r: (b, r, 0)),
                   pl.BlockSpec((1, 1, 2, dout), lambda b, r: (b, r, 0, 0))],
        out_shape=[jax.ShapeDtypeStruct((B, N, dout), jnp.float32),
                   jax.ShapeDtypeStruct((B, RC, 2, dout), jnp.float32)],
    )(s, g, dinv, bi)


def _bn(act_ref, st_ref, bng_ref, bnb_ref):
    st = st_ref[0]
    mu = jnp.sum(st[:, 0, :], axis=0, keepdims=True) * (1.0 / N)
    sq = jnp.sum(st[:, 1, :], axis=0, keepdims=True) * (1.0 / N)
    var = sq - mu * mu
    return (act_ref[0] - mu) * lax.rsqrt(var + 1e-5) * bng_ref[...] + bnb_ref[...]


def _tc_mid_body(act_ref, st_ref, dinv_ref, bng_ref, bnb_ref, w_ref, out_ref):
    h = _bn(act_ref, st_ref, bng_ref, bnb_ref)
    hw = jnp.dot(h, w_ref[...], preferred_element_type=jnp.float32)
    out_ref[0] = dinv_ref[0] * hw


def _tc_mid(act, st, dinv, bng, bnb, wn, dout, dnext):
    return pl.pallas_call(
        _tc_mid_body,
        grid=(B, RC),
        in_specs=[pl.BlockSpec((1, CHR, dout), lambda b, r: (b, r, 0)),
                  pl.BlockSpec((1, RC, 2, dout), lambda b, r: (b, 0, 0, 0)),
                  pl.BlockSpec((1, CHR, 1), lambda b, r: (b, r, 0)),
                  pl.BlockSpec((1, dout), lambda b, r: (0, 0)),
                  pl.BlockSpec((1, dout), lambda b, r: (0, 0)),
                  pl.BlockSpec((dout, dnext), lambda b, r: (0, 0))],
        out_specs=pl.BlockSpec((1, CHR, dnext), lambda b, r: (b, r, 0)),
        out_shape=jax.ShapeDtypeStruct((B, N, dnext), jnp.float32),
    )(act, st, dinv, bng, bnb, wn)


def _tc_last_body(act_ref, st_ref, bng_ref, bnb_ref, lng_ref, lnb_ref,
                  out_ref):
    h = _bn(act_ref, st_ref, bng_ref, bnb_ref)
    mu = jnp.mean(h, axis=-1, keepdims=True)
    xc = h - mu
    var = jnp.mean(xc * xc, axis=-1, keepdims=True)
    out_ref[0] = xc * lax.rsqrt(var + 1e-5) * lng_ref[...] + lnb_ref[...]


def _tc_last(act, st, bng, bnb, lng, lnb, dout):
    return pl.pallas_call(
        _tc_last_body,
        grid=(B, RC),
        in_specs=[pl.BlockSpec((1, CHR, dout), lambda b, r: (b, r, 0)),
                  pl.BlockSpec((1, RC, 2, dout), lambda b, r: (b, 0, 0, 0)),
                  pl.BlockSpec((1, dout), lambda b, r: (0, 0)),
                  pl.BlockSpec((1, dout), lambda b, r: (0, 0)),
                  pl.BlockSpec((1, dout), lambda b, r: (0, 0)),
                  pl.BlockSpec((1, dout), lambda b, r: (0, 0))],
        out_specs=pl.BlockSpec((1, CHR, dout), lambda b, r: (b, r, 0)),
        out_shape=jax.ShapeDtypeStruct((B, N, dout), jnp.float32),
    )(act, st, bng, bnb, lng, lnb)


def kernel(x, edge_index, params):
    xs = jnp.squeeze(x, -1)
    src = edge_index[:, 0, :] + (jnp.arange(B, dtype=jnp.int32) * N)[:, None]
    srcp = src.reshape(B, NW, NCHK, CH)
    dstp = edge_index[:, 1, :].reshape(B, NW, NCHK, CH)
    ones16 = jnp.ones((CH, 16), jnp.float32)
    z16 = jnp.zeros((NPAD, 16), jnp.float32)
    deg = _sc_deg(dstp, ones16, z16)
    dinv = _tc_dinv(deg)

    p = params
    vec = lambda v: v.reshape(1, -1)
    g = _tc_first(xs, dinv, p["W1"])
    for i in range(1, 6):
        dout = DOUTS[i - 1]
        zer = jnp.zeros((NPAD, dout), jnp.float32)
        s = _sc_agg(dout)(g.reshape(B * N, dout), srcp, dstp, zer)
        act, st = _tc_act(s, g, dinv, vec(p[f"b{i}"]), dout)
        if i < 5:
            g = _tc_mid(act, st, dinv, vec(p[f"bn{i}_g"]), vec(p[f"bn{i}_b"]),
                        p[f"W{i+1}"], dout, DOUTS[i])
        else:
            out = _tc_last(act, st, vec(p["bn5_g"]), vec(p["bn5_b"]),
                           vec(p["ln_g"]), vec(p["ln_b"]), dout)
    return out.reshape(B, N * DOUTS[4])
